# baseline (speedup 1.0000x reference)
import jax
import jax.numpy as jnp
from jax import lax
from jax.experimental import pallas as pl
from jax.experimental.pallas import tpu as pltpu
from jax.experimental.pallas import tpu_sc as plsc

VOCAB = 1000
EMBED_DIM = 20
BATCH = 16384
HIST = 50

NUM_WORKERS = 32
BW = BATCH // NUM_WORKERS
NGROUPS = BW // 16


def _body(idx1_hbm, idx2_hbm, table_hbm, out1_hbm, out2_hbm,
          table_v, blk1_v, blk2_v, stage_v, osem0, osem1):
    wid = lax.axis_index("s") * 2 + lax.axis_index("c")
    b0 = wid * BW
    osems = (osem0, osem1)

    pltpu.sync_copy(table_hbm, table_v)
    pltpu.sync_copy(idx1_hbm.at[pl.ds(b0 * HIST, BW * HIST)], blk1_v)
    pltpu.sync_copy(idx2_hbm.at[pl.ds(b0 * HIST, BW * HIST)], blk2_v)

    biota_h = lax.iota(jnp.int32, 16) * HIST

    def out_copy(out_hbm, h, p):
        return pltpu.make_async_copy(
            stage_v.at[p],
            out_hbm.at[pl.ds(h * EMBED_DIM, EMBED_DIM), pl.ds(b0, BW)],
            osems[p])

    def phase(blk_v, out_hbm):
        def step(i, h, p):
            @pl.when(i > 0)
            def _wo():
                out_copy(out_hbm, h, p).wait()

            def g_body(g, c):
                iv = plsc.load_gather(
                    blk_v, [biota_h + (h + g * 16 * HIST)])
                vals = [
                    plsc.load_gather(
                        table_v, [jnp.full((16,), d, jnp.int32), iv])
                    for d in range(EMBED_DIM)
                ]
                for d in range(EMBED_DIM):
                    stage_v[p, d, pl.ds(g * 16, 16)] = vals[d]
                return c
            lax.fori_loop(0, NGROUPS, g_body, 0, unroll=True)

            out_copy(out_hbm, h, p).start()

        def pair(i, carry):
            step(i, 2 * i, 0)
            step(i, 2 * i + 1, 1)
            return carry
        lax.fori_loop(0, HIST // 2, pair, 0)

        out_copy(out_hbm, HIST - 2, 0).wait()
        out_copy(out_hbm, HIST - 1, 1).wait()

    phase(blk1_v, out1_hbm)
    phase(blk2_v, out2_hbm)


def kernel(DPTD_name_1, DPTD_name_2, table):
    idx1 = DPTD_name_1.astype(jnp.int32).reshape(-1)
    idx2 = DPTD_name_2.astype(jnp.int32).reshape(-1)
    table_t = table.T

    mesh = plsc.VectorSubcoreMesh(
        core_axis_name="c", subcore_axis_name="s", num_cores=2,
        num_subcores=16)
    out_t = jax.ShapeDtypeStruct((HIST * EMBED_DIM, BATCH), jnp.float32)
    run = pl.kernel(
        _body,
        out_type=(out_t, out_t),
        mesh=mesh,
        scratch_types=[
            pltpu.VMEM((EMBED_DIM, VOCAB), jnp.float32),
            pltpu.VMEM((BW * HIST,), jnp.int32),
            pltpu.VMEM((BW * HIST,), jnp.int32),
            pltpu.VMEM((2, EMBED_DIM, BW), jnp.float32),
            pltpu.SemaphoreType.DMA,
            pltpu.SemaphoreType.DMA,
        ],
        compiler_params=pltpu.CompilerParams(
            use_tc_tiling_on_sc=False, needs_layout_passes=False),
    )
    o1, o2 = run(idx1, idx2, table_t)
    def to_logical(o):
        return o.T.reshape(1, BATCH, HIST, EMBED_DIM)
    return (to_logical(o1), to_logical(o2))


# speedup vs baseline: 1.9360x; 1.9360x over previous
import jax
import jax.numpy as jnp
from jax import lax
from jax.experimental import pallas as pl
from jax.experimental.pallas import tpu as pltpu
from jax.experimental.pallas import tpu_sc as plsc

VOCAB = 1000
EMBED_DIM = 20
BATCH = 16384
HIST = 50

NUM_WORKERS = 32
BW = BATCH // NUM_WORKERS
NGROUPS = BW // 16


def _body(idx1_hbm, idx2_hbm, table_hbm, out1_hbm, out2_hbm,
          table_v, blk1_v, blk2_v, stage_v, osem0, osem1):
    wid = lax.axis_index("s") * 2 + lax.axis_index("c")
    b0 = wid * BW
    osems = (osem0, osem1)

    pltpu.sync_copy(table_hbm, table_v)
    pltpu.sync_copy(idx1_hbm.at[pl.ds(b0 * HIST, BW * HIST)], blk1_v)
    pltpu.sync_copy(idx2_hbm.at[pl.ds(b0 * HIST, BW * HIST)], blk2_v)

    biota_h = lax.iota(jnp.int32, 16) * HIST

    def out_copy(out_hbm, h, p):
        return pltpu.make_async_copy(
            stage_v.at[p],
            out_hbm.at[pl.ds(h * EMBED_DIM, EMBED_DIM), pl.ds(b0, BW)],
            osems[p])

    def phase(blk_v, out_hbm):
        def step(i, h, p):
            @pl.when(i > 0)
            def _wo():
                out_copy(out_hbm, h, p).wait()

            PIPE = 6

            def g_body(g, c):
                iv = plsc.load_gather(
                    blk_v, [biota_h + (h + g * 16 * HIST)])
                vals = [None] * EMBED_DIM
                for d in range(EMBED_DIM):
                    vals[d] = plsc.load_gather(
                        table_v, [jnp.full((16,), d, jnp.int32), iv])
                    if d >= PIPE:
                        stage_v[p, d - PIPE, pl.ds(g * 16, 16)] = vals[d - PIPE]
                for d in range(EMBED_DIM - PIPE, EMBED_DIM):
                    stage_v[p, d, pl.ds(g * 16, 16)] = vals[d]
                return c
            lax.fori_loop(0, NGROUPS, g_body, 0, unroll=True)

            out_copy(out_hbm, h, p).start()

        def pair(i, carry):
            step(i, 2 * i, 0)
            step(i, 2 * i + 1, 1)
            return carry
        lax.fori_loop(0, HIST // 2, pair, 0)

        out_copy(out_hbm, HIST - 2, 0).wait()
        out_copy(out_hbm, HIST - 1, 1).wait()

    phase(blk1_v, out1_hbm)
    phase(blk2_v, out2_hbm)


def kernel(DPTD_name_1, DPTD_name_2, table):
    idx1 = DPTD_name_1.astype(jnp.int32).reshape(-1)
    idx2 = DPTD_name_2.astype(jnp.int32).reshape(-1)
    table_t = table.T

    mesh = plsc.VectorSubcoreMesh(
        core_axis_name="c", subcore_axis_name="s", num_cores=2,
        num_subcores=16)
    out_t = jax.ShapeDtypeStruct((HIST * EMBED_DIM, BATCH), jnp.float32)
    run = pl.kernel(
        _body,
        out_type=(out_t, out_t),
        mesh=mesh,
        scratch_types=[
            pltpu.VMEM((EMBED_DIM, VOCAB), jnp.float32),
            pltpu.VMEM((BW * HIST,), jnp.int32),
            pltpu.VMEM((BW * HIST,), jnp.int32),
            pltpu.VMEM((2, EMBED_DIM, BW), jnp.float32),
            pltpu.SemaphoreType.DMA,
            pltpu.SemaphoreType.DMA,
        ],
        compiler_params=pltpu.CompilerParams(
            use_tc_tiling_on_sc=False, needs_layout_passes=False),
    )
    o1, o2 = run(idx1, idx2, table_t)
    def to_logical(o):
        return o.reshape(HIST, EMBED_DIM, BATCH).transpose(2, 0, 1)[None]
    return (to_logical(o1), to_logical(o2))


# speedup vs baseline: 2.5799x; 1.3326x over previous
import jax
import jax.numpy as jnp
from jax import lax
from jax.experimental import pallas as pl
from jax.experimental.pallas import tpu as pltpu
from jax.experimental.pallas import tpu_sc as plsc

VOCAB = 1000
EMBED_DIM = 20
BATCH = 16384
HIST = 50

NUM_WORKERS = 32
BW = BATCH // NUM_WORKERS
NGROUPS = BW // 16


def _body(idx1_hbm, idx2_hbm, table_hbm, out1_hbm, out2_hbm,
          table_v, blk1_v, blk2_v, stage_v, osem0, osem1):
    wid = lax.axis_index("s") * 2 + lax.axis_index("c")
    b0 = wid * BW
    osems = (osem0, osem1)

    pltpu.sync_copy(table_hbm, table_v)
    pltpu.sync_copy(idx1_hbm.at[pl.ds(b0 * HIST, BW * HIST)], blk1_v)
    pltpu.sync_copy(idx2_hbm.at[pl.ds(b0 * HIST, BW * HIST)], blk2_v)

    biota_h = lax.iota(jnp.int32, 16) * HIST

    def out_copy(out_hbm, h, p):
        return pltpu.make_async_copy(
            stage_v.at[p],
            out_hbm.at[pl.ds(h * EMBED_DIM, EMBED_DIM), pl.ds(b0, BW)],
            osems[p])

    def phase(blk_v, out_hbm):
        def step(i, h, p):
            @pl.when(i > 0)
            def _wo():
                out_copy(out_hbm, h, p).wait()

            PIPE = 6

            def iv_gather(g):
                return plsc.load_gather(
                    blk_v, [biota_h + (h + g * 16 * HIST)])

            ivs = {0: iv_gather(0), 1: iv_gather(1)}
            ring = []
            for g in range(NGROUPS):
                if g + 2 < NGROUPS:
                    ivs[g + 2] = iv_gather(g + 2)
                for d in range(EMBED_DIM):
                    v = plsc.load_gather(
                        table_v, [jnp.full((16,), d, jnp.int32), ivs[g]])
                    ring.append((g, d, v))
                    if len(ring) > PIPE:
                        gg, dd, vv = ring.pop(0)
                        stage_v[p, dd, pl.ds(gg * 16, 16)] = vv
                ivs.pop(g, None)
            for gg, dd, vv in ring:
                stage_v[p, dd, pl.ds(gg * 16, 16)] = vv

            out_copy(out_hbm, h, p).start()

        def pair(i, carry):
            step(i, 2 * i, 0)
            step(i, 2 * i + 1, 1)
            return carry
        lax.fori_loop(0, HIST // 2, pair, 0)

        out_copy(out_hbm, HIST - 2, 0).wait()
        out_copy(out_hbm, HIST - 1, 1).wait()

    phase(blk1_v, out1_hbm)
    phase(blk2_v, out2_hbm)


def kernel(DPTD_name_1, DPTD_name_2, table):
    idx1 = DPTD_name_1.astype(jnp.int32).reshape(-1)
    idx2 = DPTD_name_2.astype(jnp.int32).reshape(-1)
    table_t = table.T

    mesh = plsc.VectorSubcoreMesh(
        core_axis_name="c", subcore_axis_name="s", num_cores=2,
        num_subcores=16)
    out_t = jax.ShapeDtypeStruct((HIST * EMBED_DIM, BATCH), jnp.float32)
    run = pl.kernel(
        _body,
        out_type=(out_t, out_t),
        mesh=mesh,
        scratch_types=[
            pltpu.VMEM((EMBED_DIM, VOCAB), jnp.float32),
            pltpu.VMEM((BW * HIST,), jnp.int32),
            pltpu.VMEM((BW * HIST,), jnp.int32),
            pltpu.VMEM((2, EMBED_DIM, BW), jnp.float32),
            pltpu.SemaphoreType.DMA,
            pltpu.SemaphoreType.DMA,
        ],
        compiler_params=pltpu.CompilerParams(
            use_tc_tiling_on_sc=False, needs_layout_passes=False),
    )
    o1, o2 = run(idx1, idx2, table_t)
    def to_logical(o):
        return o.reshape(HIST, EMBED_DIM, BATCH).transpose(2, 0, 1)[None]
    return (to_logical(o1), to_logical(o2))


# speedup vs baseline: 2.7925x; 1.0824x over previous
import jax
import jax.numpy as jnp
from jax import lax
from jax.experimental import pallas as pl
from jax.experimental.pallas import tpu as pltpu
from jax.experimental.pallas import tpu_sc as plsc

VOCAB = 1000
EMBED_DIM = 20
BATCH = 16384
HIST = 50

NUM_WORKERS = 32
BW = BATCH // NUM_WORKERS
NGROUPS = BW // 16


def _body(idx_hbm, table_hbm, out_hbm,
          table_v, blk_v, stage_v, osem0, osem1):
    wid = lax.axis_index("s") * 2 + lax.axis_index("c")
    b0 = wid * BW
    osems = (osem0, osem1)

    pltpu.sync_copy(table_hbm, table_v)
    pltpu.sync_copy(idx_hbm.at[pl.ds(b0 * HIST, BW * HIST)], blk_v)

    biota_h = lax.iota(jnp.int32, 16) * HIST

    def out_copy(h, p):
        return pltpu.make_async_copy(
            stage_v.at[p],
            out_hbm.at[pl.ds(h * EMBED_DIM, EMBED_DIM), pl.ds(b0, BW)],
            osems[p])

    if True:
        def step(i, h, p):
            @pl.when(i > 0)
            def _wo():
                out_copy(h, p).wait()

            PIPE = 6

            def iv_gather(g):
                return plsc.load_gather(
                    blk_v, [biota_h + (h + g * 16 * HIST)])

            ivs = {0: iv_gather(0), 1: iv_gather(1)}
            ring = []
            for g in range(NGROUPS):
                if g + 2 < NGROUPS:
                    ivs[g + 2] = iv_gather(g + 2)
                for d in range(EMBED_DIM):
                    v = plsc.load_gather(
                        table_v, [jnp.full((16,), d, jnp.int32), ivs[g]])
                    ring.append((g, d, v))
                    if len(ring) > PIPE:
                        gg, dd, vv = ring.pop(0)
                        stage_v[p, dd, pl.ds(gg * 16, 16)] = vv
                ivs.pop(g, None)
            for gg, dd, vv in ring:
                stage_v[p, dd, pl.ds(gg * 16, 16)] = vv

            out_copy(h, p).start()

        def pair(i, carry):
            step(i, 2 * i, 0)
            step(i, 2 * i + 1, 1)
            return carry
        lax.fori_loop(0, HIST // 2, pair, 0)

        out_copy(HIST - 2, 0).wait()
        out_copy(HIST - 1, 1).wait()


def kernel(DPTD_name_1, DPTD_name_2, table):
    idx1 = DPTD_name_1.astype(jnp.int32).reshape(-1)
    idx2 = DPTD_name_2.astype(jnp.int32).reshape(-1)
    table_t = table.T

    mesh = plsc.VectorSubcoreMesh(
        core_axis_name="c", subcore_axis_name="s", num_cores=2,
        num_subcores=16)
    out_t = jax.ShapeDtypeStruct((HIST * EMBED_DIM, BATCH), jnp.float32)
    run = pl.kernel(
        _body,
        out_type=out_t,
        mesh=mesh,
        scratch_types=[
            pltpu.VMEM((EMBED_DIM, VOCAB), jnp.float32),
            pltpu.VMEM((BW * HIST,), jnp.int32),
            pltpu.VMEM((2, EMBED_DIM, BW), jnp.float32),
            pltpu.SemaphoreType.DMA,
            pltpu.SemaphoreType.DMA,
        ],
        compiler_params=pltpu.CompilerParams(
            use_tc_tiling_on_sc=False, needs_layout_passes=False),
    )
    o1 = run(idx1, table_t)
    o2 = run(idx2, table_t)
    def to_logical(o):
        return o.reshape(HIST, EMBED_DIM, BATCH).transpose(2, 0, 1)[None]
    return (to_logical(o1), to_logical(o2))
